# baseline (speedup 1.0000x reference)
import functools

import jax
import jax.numpy as jnp
from jax import lax
from jax.experimental import pallas as pl
from jax.experimental.pallas import tpu as pltpu
from jax.experimental.pallas import tpu_sc as plsc

B = 4096
QL = 20
DL = 200
E = 64
NC = 2
NS = 16
NW = NC * NS
RPW = B // NW
CQ = 8
CD = 2
QS = 24
D1 = 128
D2 = DL - D1
LANES = 16
EB = E // LANES


def _pool_body(q_hbm, d1_hbm, d2_hbm, table_hbm, qout_hbm, dout_hbm,
               qidx0, qidx1, d1idx0, d1idx1, d2idx0, d2idx1,
               qrows0, qrows1, drows0, drows1,
               qacc_v, dacc_v, sem0, sem1):
    wid = lax.axis_index("s") * NC + lax.axis_index("c")
    base = wid * RPW
    sems = (sem0, sem1)

    def q_start(c, b):
        r0 = base + c * CQ
        pltpu.sync_copy(q_hbm.at[pl.ds(r0, CQ)], qidx0 if b == 0 else qidx1)
        idx = qidx0 if b == 0 else qidx1
        rows = qrows0 if b == 0 else qrows1
        for rr in range(CQ):
            pltpu.async_copy(table_hbm.at[idx.at[rr, pl.ds(0, QS)]],
                             rows.at[pl.ds(rr * QS, QS)], sems[b])

    def q_wait(b):
        idx = qidx0 if b == 0 else qidx1
        rows = qrows0 if b == 0 else qrows1
        for rr in range(CQ):
            pltpu.make_async_copy(table_hbm.at[idx.at[rr, pl.ds(0, QS)]],
                                  rows.at[pl.ds(rr * QS, QS)], sems[b]).wait()

    def q_accum(c, b):
        rows_v = qrows0 if b == 0 else qrows1
        inv = 1.0 / QL
        for rr in range(CQ):
            def seq_body(j, accs):
                row = rr * QS + j
                return tuple(accs[e] + rows_v[row, pl.ds(e * LANES, LANES)]
                             for e in range(EB))
            accs = lax.fori_loop(
                0, QL, seq_body,
                tuple(jnp.zeros((LANES,), jnp.float32) for _ in range(EB)),
                unroll=4)
            out_row = c * CQ + rr
            for e in range(EB):
                qacc_v[out_row, pl.ds(e * LANES, LANES)] = accs[e] * inv

    def d_start(c, b):
        r0 = base + c * CD
        i1 = d1idx0 if b == 0 else d1idx1
        i2 = d2idx0 if b == 0 else d2idx1
        rows = drows0 if b == 0 else drows1
        pltpu.sync_copy(d1_hbm.at[pl.ds(r0, CD)], i1)
        pltpu.sync_copy(d2_hbm.at[pl.ds(r0, CD)], i2)
        for rr in range(CD):
            pltpu.async_copy(table_hbm.at[i1.at[rr]],
                             rows.at[pl.ds(rr * DL, D1)], sems[b])
            pltpu.async_copy(table_hbm.at[i2.at[rr, pl.ds(0, D2)]],
                             rows.at[pl.ds(rr * DL + D1, D2)], sems[b])

    def d_wait(b):
        i1 = d1idx0 if b == 0 else d1idx1
        i2 = d2idx0 if b == 0 else d2idx1
        rows = drows0 if b == 0 else drows1
        for rr in range(CD):
            pltpu.make_async_copy(table_hbm.at[i1.at[rr]],
                                  rows.at[pl.ds(rr * DL, D1)], sems[b]).wait()
            pltpu.make_async_copy(table_hbm.at[i2.at[rr, pl.ds(0, D2)]],
                                  rows.at[pl.ds(rr * DL + D1, D2)], sems[b]).wait()

    def d_accum(c, b):
        rows_v = drows0 if b == 0 else drows1
        inv = 1.0 / DL
        for rr in range(CD):
            def seq_body(j, accs):
                row = rr * DL + j
                return tuple(accs[e] + rows_v[row, pl.ds(e * LANES, LANES)]
                             for e in range(EB))
            accs = lax.fori_loop(
                0, DL, seq_body,
                tuple(jnp.zeros((LANES,), jnp.float32) for _ in range(EB)),
                unroll=4)
            out_row = c * CD + rr
            for e in range(EB):
                dacc_v[out_row, pl.ds(e * LANES, LANES)] = accs[e] * inv

    def phase(start, wait, accum, nchunks):
        start(0, 0)

        @pl.loop(0, nchunks // 2)
        def _pair(p):
            c0 = 2 * p
            start(c0 + 1, 1)
            wait(0)
            accum(c0, 0)

            @pl.when(c0 + 2 < nchunks)
            def _prefetch():
                start(c0 + 2, 0)

            wait(1)
            accum(c0 + 1, 1)

    phase(q_start, q_wait, q_accum, RPW // CQ)
    phase(d_start, d_wait, d_accum, RPW // CD)
    pltpu.sync_copy(qacc_v, qout_hbm.at[pl.ds(base, RPW)])
    pltpu.sync_copy(dacc_v, dout_hbm.at[pl.ds(base, RPW)])


@functools.lru_cache(maxsize=None)
def _pool_kernel():
    return functools.partial(
        pl.kernel,
        out_type=(jax.ShapeDtypeStruct((B, E), jnp.float32),
                  jax.ShapeDtypeStruct((B, E), jnp.float32)),
        mesh=plsc.VectorSubcoreMesh(core_axis_name="c", subcore_axis_name="s",
                                    num_cores=NC, num_subcores=NS),
        scratch_types=[
            pltpu.VMEM((CQ, 128), jnp.int32),
            pltpu.VMEM((CQ, 128), jnp.int32),
            pltpu.VMEM((CD, 128), jnp.int32),
            pltpu.VMEM((CD, 128), jnp.int32),
            pltpu.VMEM((CD, 128), jnp.int32),
            pltpu.VMEM((CD, 128), jnp.int32),
            pltpu.VMEM((CQ * QS, E), jnp.float32),
            pltpu.VMEM((CQ * QS, E), jnp.float32),
            pltpu.VMEM((CD * DL, E), jnp.float32),
            pltpu.VMEM((CD * DL, E), jnp.float32),
            pltpu.VMEM((RPW, E), jnp.float32),
            pltpu.VMEM((RPW, E), jnp.float32),
            pltpu.SemaphoreType.DMA,
            pltpu.SemaphoreType.DMA,
        ],
        compiler_params=pltpu.CompilerParams(use_tc_tiling_on_sc=False),
    )(_pool_body)


def _head_body(q_ref, d_ref, wq0, bq0, wqv, bqv, wd0, bd0, wdv, bdv, wo, bo,
               out_ref, cos_ref):
    fq = jnp.dot(wq0[...], wqv[...], preferred_element_type=jnp.float32)
    bq = jnp.dot(bq0[...], wqv[...], preferred_element_type=jnp.float32) + bqv[...]
    fd = jnp.dot(wd0[...], wdv[...], preferred_element_type=jnp.float32)
    bd = jnp.dot(bd0[...], wdv[...], preferred_element_type=jnp.float32) + bdv[...]
    qv = jnp.dot(q_ref[...], fq, preferred_element_type=jnp.float32) + bq
    dv = jnp.dot(d_ref[...], fd, preferred_element_type=jnp.float32) + bd
    qn = qv / jnp.sqrt(jnp.maximum(jnp.sum(qv * qv, axis=-1, keepdims=True), 1e-12))
    dn = dv / jnp.sqrt(jnp.maximum(jnp.sum(dv * dv, axis=-1, keepdims=True), 1e-12))
    cos = jnp.sum(qn * dn, axis=-1, keepdims=True)
    cos_ref[...] = cos
    out_ref[...] = jax.nn.sigmoid(cos * wo[0, 0] + bo[0, 0])


_head_call = pl.pallas_call(
    _head_body,
    out_shape=(jax.ShapeDtypeStruct((B, 1), jnp.float32),
               jax.ShapeDtypeStruct((B, 1), jnp.float32)),
)


def kernel(query, doc, table, Wq0, bq0, Wqv, bqv, Wd0, bd0, Wdv, bdv, Wo, bo):
    q_p = jnp.pad(query, ((0, 0), (0, 128 - QL)))
    d1 = doc[:, :D1]
    d2 = jnp.pad(doc[:, D1:], ((0, 0), (0, 128 - D2)))
    q_emb, d_emb = _pool_kernel()(q_p, d1, d2, table)
    out, cos = _head_call(q_emb, d_emb,
                          Wq0, bq0.reshape(1, -1), Wqv, bqv.reshape(1, -1),
                          Wd0, bd0.reshape(1, -1), Wdv, bdv.reshape(1, -1),
                          Wo, bo.reshape(1, 1))
    return (out, cos)


# speedup vs baseline: 1.3967x; 1.3967x over previous
import functools

import jax
import jax.numpy as jnp
from jax import lax
from jax.experimental import pallas as pl
from jax.experimental.pallas import tpu as pltpu
from jax.experimental.pallas import tpu_sc as plsc

B = 4096
QL = 20
DL = 200
E = 64
W = 128
NC = 2
NS = 16
NW = NC * NS
RPW = B // NW
CQ = 8
CD = 1
LANES = 16
EB = E // LANES


def _pool_body(q_hbm, d_hbm, table_hbm, qout_hbm, dout_hbm,
               qidx0, qidx1, didx0, didx1, qrows0, qrows1, drows0, drows1,
               qacc_v, dacc_v, sem0, sem1):
    wid = lax.axis_index("s") * NC + lax.axis_index("c")
    base = wid * RPW
    sems = (sem0, sem1)

    def phase(idx_hbm, L, rows_per_chunk, idxs, rowss, acc_v):
        nchunks = RPW // rows_per_chunk
        inv = 1.0 / L

        def start(c, b):
            r0 = base + c * rows_per_chunk
            pltpu.sync_copy(idx_hbm.at[pl.ds(r0, rows_per_chunk)], idxs[b])
            for rr in range(rows_per_chunk):
                pltpu.async_copy(table_hbm.at[idxs[b].at[rr]],
                                 rowss[b].at[pl.ds(rr * L, L)], sems[b])

        def wait(b):
            for rr in range(rows_per_chunk):
                pltpu.make_async_copy(table_hbm.at[idxs[b].at[rr]],
                                      rowss[b].at[pl.ds(rr * L, L)],
                                      sems[b]).wait()

        def accum(c, b):
            rows_v = rowss[b]
            for rr in range(rows_per_chunk):
                def seq_body(j, accs):
                    row = rr * L + j
                    return tuple(accs[e] + rows_v[row, pl.ds(e * LANES, LANES)]
                                 for e in range(EB))
                accs = lax.fori_loop(
                    0, L, seq_body,
                    tuple(jnp.zeros((LANES,), jnp.float32) for _ in range(EB)),
                    unroll=4)
                out_row = c * rows_per_chunk + rr
                for e in range(EB):
                    acc_v[out_row, pl.ds(e * LANES, LANES)] = accs[e] * inv

        start(0, 0)

        @pl.loop(0, nchunks // 2)
        def _pair(p):
            c0 = 2 * p
            start(c0 + 1, 1)
            wait(0)
            accum(c0, 0)

            @pl.when(c0 + 2 < nchunks)
            def _prefetch():
                start(c0 + 2, 0)

            wait(1)
            accum(c0 + 1, 1)

    phase(q_hbm, QL, CQ, (qidx0, qidx1), (qrows0, qrows1), qacc_v)
    phase(d_hbm, DL, CD, (didx0, didx1), (drows0, drows1), dacc_v)
    pltpu.sync_copy(qacc_v, qout_hbm.at[pl.ds(base, RPW)])
    pltpu.sync_copy(dacc_v, dout_hbm.at[pl.ds(base, RPW)])


@functools.lru_cache(maxsize=None)
def _pool_kernel():
    return functools.partial(
        pl.kernel,
        out_type=(jax.ShapeDtypeStruct((B, E), jnp.float32),
                  jax.ShapeDtypeStruct((B, E), jnp.float32)),
        mesh=plsc.VectorSubcoreMesh(core_axis_name="c", subcore_axis_name="s",
                                    num_cores=NC, num_subcores=NS),
        scratch_types=[
            pltpu.VMEM((CQ, QL), jnp.int32),
            pltpu.VMEM((CQ, QL), jnp.int32),
            pltpu.VMEM((CD, DL), jnp.int32),
            pltpu.VMEM((CD, DL), jnp.int32),
            pltpu.VMEM((CQ * QL, W), jnp.float32),
            pltpu.VMEM((CQ * QL, W), jnp.float32),
            pltpu.VMEM((CD * DL, W), jnp.float32),
            pltpu.VMEM((CD * DL, W), jnp.float32),
            pltpu.VMEM((RPW, E), jnp.float32),
            pltpu.VMEM((RPW, E), jnp.float32),
            pltpu.SemaphoreType.DMA,
            pltpu.SemaphoreType.DMA,
        ],
        compiler_params=pltpu.CompilerParams(use_tc_tiling_on_sc=False),
    )(_pool_body)


def _head_body(q_ref, d_ref, wq0, bq0, wqv, bqv, wd0, bd0, wdv, bdv, wo, bo,
               out_ref, cos_ref):
    fq = jnp.dot(wq0[...], wqv[...], preferred_element_type=jnp.float32)
    bq = jnp.dot(bq0[...], wqv[...], preferred_element_type=jnp.float32) + bqv[...]
    fd = jnp.dot(wd0[...], wdv[...], preferred_element_type=jnp.float32)
    bd = jnp.dot(bd0[...], wdv[...], preferred_element_type=jnp.float32) + bdv[...]
    qv = jnp.dot(q_ref[...], fq, preferred_element_type=jnp.float32) + bq
    dv = jnp.dot(d_ref[...], fd, preferred_element_type=jnp.float32) + bd
    qn = qv / jnp.sqrt(jnp.maximum(jnp.sum(qv * qv, axis=-1, keepdims=True), 1e-12))
    dn = dv / jnp.sqrt(jnp.maximum(jnp.sum(dv * dv, axis=-1, keepdims=True), 1e-12))
    cos = jnp.sum(qn * dn, axis=-1, keepdims=True)
    cos_ref[...] = cos
    out_ref[...] = jax.nn.sigmoid(cos * wo[0, 0] + bo[0, 0])


_head_call = pl.pallas_call(
    _head_body,
    out_shape=(jax.ShapeDtypeStruct((B, 1), jnp.float32),
               jax.ShapeDtypeStruct((B, 1), jnp.float32)),
)


def kernel(query, doc, table, Wq0, bq0, Wqv, bqv, Wd0, bd0, Wdv, bdv, Wo, bo):
    table_pad = jnp.pad(table, ((0, 0), (0, W - E)))
    q_emb, d_emb = _pool_kernel()(query, doc, table_pad)
    out, cos = _head_call(q_emb, d_emb,
                          Wq0, bq0.reshape(1, -1), Wqv, bqv.reshape(1, -1),
                          Wd0, bd0.reshape(1, -1), Wdv, bdv.reshape(1, -1),
                          Wo, bo.reshape(1, 1))
    return (out, cos)
